# baseline (speedup 1.0000x reference)
import functools

import jax
import jax.numpy as jnp
from jax import lax
from jax.experimental import pallas as pl
from jax.experimental.pallas import tpu as pltpu
from jax.experimental.pallas import tpu_sc as plsc

B = 4096
L = 50
D = 128
TYPE_VOCAB = 9
REF_VOCAB = 51

_info = plsc.get_sparse_core_info()
NC = _info.num_cores
NS = _info.num_subcores
NW = NC * NS
C = 400
G = C // 16
ROWS_PER_CHUNK = C // L

K = 2
BK = B // K

_mesh = plsc.VectorSubcoreMesh(core_axis_name="c", subcore_axis_name="s")


def _build(bk):
    n = bk * L
    tok_per_w = n // NW
    nchunk = tok_per_w // C
    half = nchunk // 2
    rows_per_w = tok_per_w // L

    @functools.partial(
        pl.kernel,
        mesh=_mesh,
        out_type=jax.ShapeDtypeStruct((bk, L, D), jnp.float32),
        scratch_types=[
            pltpu.VMEM((tok_per_w,), jnp.int32),
            pltpu.VMEM((tok_per_w,), jnp.int32),
            pltpu.VMEM((tok_per_w,), jnp.int32),
            pltpu.VMEM((C, D), jnp.float32),
            pltpu.VMEM((C, D), jnp.float32),
            pltpu.VMEM((TYPE_VOCAB, D), jnp.float32),
            pltpu.VMEM((REF_VOCAB, D), jnp.float32),
            pltpu.SemaphoreType.DMA,
            pltpu.SemaphoreType.DMA,
            pltpu.SemaphoreType.DMA,
            pltpu.SemaphoreType.DMA,
        ],
    )
    def _cad_embed(tids, pids, rids, ttab, ptab, rtab, out,
                   pidx_a, tidx_a, ridx_a, prow0, prow1,
                   ttab_v, rtab_v, sg0, sg1, so0, so1):
        wid = lax.axis_index("s") * NC + lax.axis_index("c")
        base = wid * tok_per_w
        brow_base = wid * rows_per_w
        prow = (prow0, prow1)
        sg = (sg0, sg1)
        so = (so0, so1)

        pltpu.sync_copy(ttab, ttab_v)
        pltpu.sync_copy(rtab, rtab_v)

        def drain_out(buf, sem):
            for r in range(ROWS_PER_CHUNK):
                pltpu.make_async_copy(
                    buf.at[pl.ds(r * L, L)], out.at[brow_base], sem).wait()

        pltpu.sync_copy(pids.at[pl.ds(base, tok_per_w)], pidx_a)
        pltpu.async_copy(ptab.at[pidx_a.at[pl.ds(0, C)]], prow0, sg0)
        pltpu.sync_copy(tids.at[pl.ds(base, tok_per_w)], tidx_a)
        pltpu.sync_copy(rids.at[pl.ds(base, tok_per_w)], ridx_a)

        def iter_body(i, carry):
            for b in range(2):
                k = 2 * i + b
                off = k * C
                nb = 1 - b

                if b == 0:
                    @pl.when(i >= 1)
                    def _():
                        drain_out(prow[nb], so[nb])

                    pltpu.async_copy(
                        ptab.at[pidx_a.at[pl.ds(off + C, C)]],
                        prow[nb], sg[nb])
                else:
                    @pl.when(i < half - 1)
                    def _():
                        drain_out(prow[nb], so[nb])
                        pltpu.async_copy(
                            ptab.at[pidx_a.at[pl.ds(off + C, C)]],
                            prow[nb], sg[nb])

                pltpu.make_async_copy(
                    ptab.at[pl.ds(0, C)], prow[b], sg[b]).wait()
                prow_b = prow[b]

                @plsc.parallel_loop(0, G)
                def _(g):
                    tv = tidx_a[pl.ds(off + g * 16, 16)]
                    rv = ridx_a[pl.ds(off + g * 16, 16)]
                    for j in range(16):
                        row = g * 16 + j
                        ts = tv[j]
                        rs = rv[j]
                        for cb in range(D // 16):
                            sl = pl.ds(cb * 16, 16)
                            plsc.addupdate(prow_b.at[row, sl],
                                           ttab_v[ts, sl] + rtab_v[rs, sl])

                brow0 = brow_base + k * ROWS_PER_CHUNK
                for r in range(ROWS_PER_CHUNK):
                    pltpu.async_copy(
                        prow_b.at[pl.ds(r * L, L)], out.at[brow0 + r], so[b])
            return carry

        lax.fori_loop(0, half, iter_body, 0)

        drain_out(prow0, so0)
        drain_out(prow1, so1)

    return _cad_embed


_embed_k = _build(BK)


def kernel(type_ids, posi_ids, ref_ids, type_table, posi_table, ref_table):
    outs = []
    for s in range(K):
        sl = slice(s * BK, (s + 1) * BK)
        outs.append(_embed_k(
            type_ids[sl].reshape(BK * L),
            posi_ids[sl].reshape(BK * L),
            ref_ids[sl].reshape(BK * L),
            type_table,
            posi_table,
            ref_table,
        ))
    return jnp.concatenate(outs, axis=0)


# speedup vs baseline: 1.4448x; 1.4448x over previous
import functools

import jax
import jax.numpy as jnp
from jax import lax
from jax.experimental import pallas as pl
from jax.experimental.pallas import tpu as pltpu
from jax.experimental.pallas import tpu_sc as plsc

B = 4096
L = 50
D = 128
TYPE_VOCAB = 9
REF_VOCAB = 51

_info = plsc.get_sparse_core_info()
NC = _info.num_cores
NS = _info.num_subcores
NW = NC * NS
C = 400
G = C // 16
ROWS_PER_CHUNK = C // L

K = 1
BK = B // K

_mesh = plsc.VectorSubcoreMesh(core_axis_name="c", subcore_axis_name="s")


def _build(bk):
    n = bk * L
    tok_per_w = n // NW
    nchunk = tok_per_w // C
    half = nchunk // 2
    rows_per_w = tok_per_w // L

    @functools.partial(
        pl.kernel,
        mesh=_mesh,
        out_type=jax.ShapeDtypeStruct((bk, L, D), jnp.float32),
        scratch_types=[
            pltpu.VMEM((tok_per_w,), jnp.int32),
            pltpu.VMEM((tok_per_w,), jnp.int32),
            pltpu.VMEM((tok_per_w,), jnp.int32),
            pltpu.VMEM((C, D), jnp.float32),
            pltpu.VMEM((C, D), jnp.float32),
            pltpu.VMEM((TYPE_VOCAB, D), jnp.float32),
            pltpu.VMEM((REF_VOCAB, D), jnp.float32),
            pltpu.SemaphoreType.DMA,
            pltpu.SemaphoreType.DMA,
            pltpu.SemaphoreType.DMA,
            pltpu.SemaphoreType.DMA,
        ],
    )
    def _cad_embed(tids, pids, rids, ttab, ptab, rtab, out,
                   pidx_a, tidx_a, ridx_a, prow0, prow1,
                   ttab_v, rtab_v, sg0, sg1, so0, so1):
        wid = lax.axis_index("s") * NC + lax.axis_index("c")
        base = wid * tok_per_w
        brow_base = wid * rows_per_w
        prow = (prow0, prow1)
        sg = (sg0, sg1)
        so = (so0, so1)

        pltpu.sync_copy(ttab, ttab_v)
        pltpu.sync_copy(rtab, rtab_v)

        def drain_out(buf, sem):
            for r in range(ROWS_PER_CHUNK):
                pltpu.make_async_copy(
                    buf.at[pl.ds(r * L, L)], out.at[brow_base], sem).wait()

        pltpu.sync_copy(pids.at[pl.ds(base, tok_per_w)], pidx_a)
        pltpu.async_copy(ptab.at[pidx_a.at[pl.ds(0, C)]], prow0, sg0)
        pltpu.sync_copy(tids.at[pl.ds(base, tok_per_w)], tidx_a)
        pltpu.sync_copy(rids.at[pl.ds(base, tok_per_w)], ridx_a)

        def iter_body(i, carry):
            for b in range(2):
                k = 2 * i + b
                off = k * C
                nb = 1 - b

                if b == 0:
                    @pl.when(i >= 1)
                    def _():
                        drain_out(prow[nb], so[nb])

                    pltpu.async_copy(
                        ptab.at[pidx_a.at[pl.ds(off + C, C)]],
                        prow[nb], sg[nb])
                else:
                    @pl.when(i < half - 1)
                    def _():
                        drain_out(prow[nb], so[nb])
                        pltpu.async_copy(
                            ptab.at[pidx_a.at[pl.ds(off + C, C)]],
                            prow[nb], sg[nb])

                pltpu.make_async_copy(
                    ptab.at[pl.ds(0, C)], prow[b], sg[b]).wait()
                prow_b = prow[b]

                @plsc.parallel_loop(0, G)
                def _(g):
                    tv = tidx_a[pl.ds(off + g * 16, 16)]
                    rv = ridx_a[pl.ds(off + g * 16, 16)]
                    for j in range(16):
                        row = g * 16 + j
                        ts = tv[j]
                        rs = rv[j]
                        for cb in range(D // 16):
                            sl = pl.ds(cb * 16, 16)
                            plsc.addupdate(prow_b.at[row, sl],
                                           ttab_v[ts, sl] + rtab_v[rs, sl])

                brow0 = brow_base + k * ROWS_PER_CHUNK
                for r in range(ROWS_PER_CHUNK):
                    pltpu.async_copy(
                        prow_b.at[pl.ds(r * L, L)], out.at[brow0 + r], so[b])
            return carry

        lax.fori_loop(0, half, iter_body, 0)

        drain_out(prow0, so0)
        drain_out(prow1, so1)

    return _cad_embed


_embed_k = _build(BK)


def kernel(type_ids, posi_ids, ref_ids, type_table, posi_table, ref_table):
    outs = []
    for s in range(K):
        sl = slice(s * BK, (s + 1) * BK)
        outs.append(_embed_k(
            type_ids[sl].reshape(BK * L),
            posi_ids[sl].reshape(BK * L),
            ref_ids[sl].reshape(BK * L),
            type_table,
            posi_table,
            ref_table,
        ))
    if K == 1:
        return outs[0]
    return jnp.concatenate(outs, axis=0)
